# Initial kernel scaffold; baseline (speedup 1.0000x reference)
import jax
import jax.numpy as jnp
from jax.experimental import pallas as pl


def kernel(posterior_probabilities, support, prior_probabilities):
    raise NotImplementedError("write your pallas kernel here")


# speedup vs baseline: 2.9173x; 2.9173x over previous
import jax
import jax.numpy as jnp
from jax.experimental import pallas as pl
from jax.experimental.pallas import tpu as pltpu


def _fused_body(post_ref, sup_ref, prior_ref, ms_ref, pe_ref, mi_ref):
    p = post_ref[0]
    sup = sup_ref[0]
    prior = prior_ref[0]
    k = p.shape[-1]

    logp = jnp.log(jnp.maximum(p, 1e-12))
    ent = -jnp.sum(p * logp, axis=-1)

    m = jnp.max(p, axis=-1, keepdims=True)
    iota = jax.lax.broadcasted_iota(jnp.int32, p.shape, 1)
    idx = jnp.min(jnp.where(p == m, iota, k), axis=-1)
    ms = jnp.sum(jnp.where(iota == idx[:, None], sup, 0.0), axis=-1)

    prior_ent = -jnp.sum(prior * jnp.log(jnp.maximum(prior, 1e-12)))
    mi = jnp.maximum(prior_ent - ent, 0.0)

    ms_ref[0, 0] = ms
    pe_ref[0, 0] = ent
    mi_ref[0, 0] = mi


def kernel(posterior_probabilities, support, prior_probabilities):
    b, g, k = posterior_probabilities.shape
    gt = 256
    ng = g // gt

    sup3 = support.reshape(b, 1, k)
    prior3 = prior_probabilities.reshape(b, 1, k)

    out_sd = jax.ShapeDtypeStruct((b * ng, 1, gt), jnp.float32)
    out_spec = pl.BlockSpec((1, 1, gt), lambda i, j: (i * ng + j, 0, 0))
    ms, pe, mi = pl.pallas_call(
        _fused_body,
        grid=(b, ng),
        in_specs=[
            pl.BlockSpec((1, gt, k), lambda i, j: (i, j, 0)),
            pl.BlockSpec((1, 1, k), lambda i, j: (i, 0, 0)),
            pl.BlockSpec((1, 1, k), lambda i, j: (i, 0, 0)),
        ],
        out_specs=[out_spec, out_spec, out_spec],
        out_shape=[out_sd, out_sd, out_sd],
        compiler_params=pltpu.CompilerParams(
            dimension_semantics=("parallel", "parallel"),
        ),
    )(posterior_probabilities, sup3, prior3)

    return (ms.reshape(b, g), pe.reshape(b, g), mi.reshape(b, g))


# speedup vs baseline: 3.5097x; 1.2031x over previous
import jax
import jax.numpy as jnp
from jax.experimental import pallas as pl
from jax.experimental.pallas import tpu as pltpu


def _fused_body(post_ref, sup_ref, prior_ref, ms_ref, pe_ref, mi_ref):
    p = post_ref[0]
    sup = sup_ref[0]
    prior = prior_ref[0]
    k = p.shape[-1]

    pt = jax.lax.transpose(p, (1, 0))
    logp = jnp.log(jnp.maximum(pt, 1e-12))
    ent = -jnp.sum(pt * logp, axis=0, keepdims=True)

    m = jnp.max(pt, axis=0, keepdims=True)
    iota = jax.lax.broadcasted_iota(jnp.int32, pt.shape, 0)
    idx = jnp.min(jnp.where(pt == m, iota, k), axis=0, keepdims=True)
    supt = jax.lax.transpose(sup, (1, 0))
    ms = jnp.sum(jnp.where(iota == idx, supt, 0.0), axis=0, keepdims=True)

    prior_ent = -jnp.sum(prior * jnp.log(jnp.maximum(prior, 1e-12)))
    mi = jnp.maximum(prior_ent - ent, 0.0)

    ms_ref[0] = ms
    pe_ref[0] = ent
    mi_ref[0] = mi


def kernel(posterior_probabilities, support, prior_probabilities):
    b, g, k = posterior_probabilities.shape
    gt = 256
    ng = g // gt

    sup3 = support.reshape(b, 1, k)
    prior3 = prior_probabilities.reshape(b, 1, k)

    out_sd = jax.ShapeDtypeStruct((b * ng, 1, gt), jnp.float32)
    out_spec = pl.BlockSpec((1, 1, gt), lambda i, j: (i * ng + j, 0, 0))
    ms, pe, mi = pl.pallas_call(
        _fused_body,
        grid=(b, ng),
        in_specs=[
            pl.BlockSpec((1, gt, k), lambda i, j: (i, j, 0)),
            pl.BlockSpec((1, 1, k), lambda i, j: (i, 0, 0)),
            pl.BlockSpec((1, 1, k), lambda i, j: (i, 0, 0)),
        ],
        out_specs=[out_spec, out_spec, out_spec],
        out_shape=[out_sd, out_sd, out_sd],
        compiler_params=pltpu.CompilerParams(
            dimension_semantics=("parallel", "parallel"),
        ),
    )(posterior_probabilities, sup3, prior3)

    return (ms.reshape(b, g), pe.reshape(b, g), mi.reshape(b, g))


# speedup vs baseline: 5.3173x; 1.5150x over previous
import jax
import jax.numpy as jnp
from jax.experimental import pallas as pl
from jax.experimental.pallas import tpu as pltpu


def _fused_body(post_ref, sup_ref, prior_ref, ms_ref, pe_ref, mi_ref):
    p = post_ref[0]
    sup = sup_ref[0]
    prior = prior_ref[0]
    k = p.shape[-1]

    pt = jax.lax.transpose(p, (1, 0))
    logp = jnp.log(jnp.maximum(pt, 1e-12))
    ent = -jnp.sum(pt * logp, axis=0, keepdims=True)

    m = jnp.max(pt, axis=0, keepdims=True)
    iota = jax.lax.broadcasted_iota(jnp.int32, pt.shape, 0)
    idx = jnp.min(jnp.where(pt == m, iota, k), axis=0, keepdims=True)
    supt = jax.lax.transpose(sup, (1, 0))
    ms = jnp.sum(jnp.where(iota == idx, supt, 0.0), axis=0, keepdims=True)

    prior_ent = -jnp.sum(prior * jnp.log(jnp.maximum(prior, 1e-12)))
    mi = jnp.maximum(prior_ent - ent, 0.0)

    ms_ref[0] = ms
    pe_ref[0] = ent
    mi_ref[0] = mi


def kernel(posterior_probabilities, support, prior_probabilities):
    b, g, k = posterior_probabilities.shape
    gt = 512
    ng = g // gt

    sup3 = support.reshape(b, 1, k)
    prior3 = prior_probabilities.reshape(b, 1, k)

    out_sd = jax.ShapeDtypeStruct((b * ng, 1, gt), jnp.float32)
    out_spec = pl.BlockSpec((1, 1, gt), lambda i, j: (i * ng + j, 0, 0))
    ms, pe, mi = pl.pallas_call(
        _fused_body,
        grid=(b, ng),
        in_specs=[
            pl.BlockSpec((1, gt, k), lambda i, j: (i, j, 0)),
            pl.BlockSpec((1, 1, k), lambda i, j: (i, 0, 0)),
            pl.BlockSpec((1, 1, k), lambda i, j: (i, 0, 0)),
        ],
        out_specs=[out_spec, out_spec, out_spec],
        out_shape=[out_sd, out_sd, out_sd],
        compiler_params=pltpu.CompilerParams(
            dimension_semantics=("parallel", "parallel"),
        ),
    )(posterior_probabilities, sup3, prior3)

    return (ms.reshape(b, g), pe.reshape(b, g), mi.reshape(b, g))


# speedup vs baseline: 7.3231x; 1.3772x over previous
import jax
import jax.numpy as jnp
from jax.experimental import pallas as pl
from jax.experimental.pallas import tpu as pltpu


def _fused_body(post_ref, sup_ref, prior_ref, ms_ref, pe_ref, mi_ref):
    p = post_ref[0]
    sup = sup_ref[0]
    prior = prior_ref[0]
    k = p.shape[-1]

    pt = jax.lax.transpose(p, (1, 0))
    logp = jnp.log(jnp.maximum(pt, 1e-12))
    ent = -jnp.sum(pt * logp, axis=0, keepdims=True)

    m = jnp.max(pt, axis=0, keepdims=True)
    iota = jax.lax.broadcasted_iota(jnp.int32, pt.shape, 0)
    idx = jnp.min(jnp.where(pt == m, iota, k), axis=0, keepdims=True)
    supt = jax.lax.transpose(sup, (1, 0))
    ms = jnp.sum(jnp.where(iota == idx, supt, 0.0), axis=0, keepdims=True)

    prior_ent = -jnp.sum(prior * jnp.log(jnp.maximum(prior, 1e-12)))
    mi = jnp.maximum(prior_ent - ent, 0.0)

    ms_ref[0] = ms
    pe_ref[0] = ent
    mi_ref[0] = mi


def kernel(posterior_probabilities, support, prior_probabilities):
    b, g, k = posterior_probabilities.shape
    gt = 1024
    ng = g // gt

    sup3 = support.reshape(b, 1, k)
    prior3 = prior_probabilities.reshape(b, 1, k)

    out_sd = jax.ShapeDtypeStruct((b * ng, 1, gt), jnp.float32)
    out_spec = pl.BlockSpec((1, 1, gt), lambda i, j: (i * ng + j, 0, 0))
    ms, pe, mi = pl.pallas_call(
        _fused_body,
        grid=(b, ng),
        in_specs=[
            pl.BlockSpec((1, gt, k), lambda i, j: (i, j, 0)),
            pl.BlockSpec((1, 1, k), lambda i, j: (i, 0, 0)),
            pl.BlockSpec((1, 1, k), lambda i, j: (i, 0, 0)),
        ],
        out_specs=[out_spec, out_spec, out_spec],
        out_shape=[out_sd, out_sd, out_sd],
        compiler_params=pltpu.CompilerParams(
            dimension_semantics=("parallel", "parallel"),
        ),
    )(posterior_probabilities, sup3, prior3)

    return (ms.reshape(b, g), pe.reshape(b, g), mi.reshape(b, g))


# speedup vs baseline: 8.6587x; 1.1824x over previous
import jax
import jax.numpy as jnp
from jax.experimental import pallas as pl
from jax.experimental.pallas import tpu as pltpu

_BT = 2


def _fused_body(post_ref, sup_ref, prior_ref, ms_ref, pe_ref, mi_ref):
    k = post_ref.shape[-1]
    for bb in range(_BT):
        p = post_ref[bb]
        sup = sup_ref[bb]
        prior = prior_ref[bb]

        pt = jax.lax.transpose(p, (1, 0))
        logp = jnp.log(jnp.maximum(pt, 1e-12))
        ent = -jnp.sum(pt * logp, axis=0, keepdims=True)

        m = jnp.max(pt, axis=0, keepdims=True)
        iota = jax.lax.broadcasted_iota(jnp.int32, pt.shape, 0)
        idx = jnp.min(jnp.where(pt == m, iota, k), axis=0, keepdims=True)
        supt = jax.lax.transpose(sup, (1, 0))
        ms = jnp.sum(jnp.where(iota == idx, supt, 0.0), axis=0, keepdims=True)

        prior_ent = -jnp.sum(prior * jnp.log(jnp.maximum(prior, 1e-12)))
        mi = jnp.maximum(prior_ent - ent, 0.0)

        ms_ref[bb] = ms
        pe_ref[bb] = ent
        mi_ref[bb] = mi


def kernel(posterior_probabilities, support, prior_probabilities):
    b, g, k = posterior_probabilities.shape
    nb = b // _BT

    sup3 = support.reshape(b, 1, k)
    prior3 = prior_probabilities.reshape(b, 1, k)

    out_sd = jax.ShapeDtypeStruct((b, 1, g), jnp.float32)
    out_spec = pl.BlockSpec((_BT, 1, g), lambda i: (i, 0, 0))
    ms, pe, mi = pl.pallas_call(
        _fused_body,
        grid=(nb,),
        in_specs=[
            pl.BlockSpec((_BT, g, k), lambda i: (i, 0, 0)),
            pl.BlockSpec((_BT, 1, k), lambda i: (i, 0, 0)),
            pl.BlockSpec((_BT, 1, k), lambda i: (i, 0, 0)),
        ],
        out_specs=[out_spec, out_spec, out_spec],
        out_shape=[out_sd, out_sd, out_sd],
        compiler_params=pltpu.CompilerParams(
            dimension_semantics=("parallel",),
        ),
    )(posterior_probabilities, sup3, prior3)

    return (ms.reshape(b, g), pe.reshape(b, g), mi.reshape(b, g))


# speedup vs baseline: 8.6804x; 1.0025x over previous
import jax
import jax.numpy as jnp
from jax.experimental import pallas as pl
from jax.experimental.pallas import tpu as pltpu

_BT = 4


def _fused_body(post_ref, sup_ref, prior_ref, ms_ref, pe_ref, mi_ref):
    k = post_ref.shape[-1]
    for bb in range(_BT):
        p = post_ref[bb]
        sup = sup_ref[bb]
        prior = prior_ref[bb]

        pt = jax.lax.transpose(p, (1, 0))
        logp = jnp.log(jnp.maximum(pt, 1e-12))
        ent = -jnp.sum(pt * logp, axis=0, keepdims=True)

        m = jnp.max(pt, axis=0, keepdims=True)
        iota = jax.lax.broadcasted_iota(jnp.int32, pt.shape, 0)
        idx = jnp.min(jnp.where(pt == m, iota, k), axis=0, keepdims=True)
        supt = jax.lax.transpose(sup, (1, 0))
        ms = jnp.sum(jnp.where(iota == idx, supt, 0.0), axis=0, keepdims=True)

        prior_ent = -jnp.sum(prior * jnp.log(jnp.maximum(prior, 1e-12)))
        mi = jnp.maximum(prior_ent - ent, 0.0)

        ms_ref[bb] = ms
        pe_ref[bb] = ent
        mi_ref[bb] = mi


def kernel(posterior_probabilities, support, prior_probabilities):
    b, g, k = posterior_probabilities.shape
    nb = b // _BT

    sup3 = support.reshape(b, 1, k)
    prior3 = prior_probabilities.reshape(b, 1, k)

    out_sd = jax.ShapeDtypeStruct((b, 1, g), jnp.float32)
    out_spec = pl.BlockSpec((_BT, 1, g), lambda i: (i, 0, 0))
    ms, pe, mi = pl.pallas_call(
        _fused_body,
        grid=(nb,),
        in_specs=[
            pl.BlockSpec((_BT, g, k), lambda i: (i, 0, 0)),
            pl.BlockSpec((_BT, 1, k), lambda i: (i, 0, 0)),
            pl.BlockSpec((_BT, 1, k), lambda i: (i, 0, 0)),
        ],
        out_specs=[out_spec, out_spec, out_spec],
        out_shape=[out_sd, out_sd, out_sd],
        compiler_params=pltpu.CompilerParams(
            dimension_semantics=("parallel",),
        ),
    )(posterior_probabilities, sup3, prior3)

    return (ms.reshape(b, g), pe.reshape(b, g), mi.reshape(b, g))
